# baseline (speedup 1.0000x reference)
import functools

import jax
import jax.numpy as jnp
from jax import lax
from jax.experimental import pallas as pl
from jax.experimental.pallas import tpu as pltpu
from jax.experimental.pallas import tpu_sc as plsc

OUTPUT_DIM = 32
NUM_FIELDS = 6
FIELD_SIZE = 100
BATCH = 16384
VOCAB = NUM_FIELDS * FIELD_SIZE
PACK = 128 // OUTPUT_DIM
TROWS = VOCAB // PACK

NC, NS, L = 2, 16, 16
NW = NC * NS
B_PER_W = BATCH // NW
BLK = 128
NBLK = B_PER_W // BLK
GROUPS = BLK // L

_mesh = plsc.VectorSubcoreMesh(
    core_axis_name="c", subcore_axis_name="s", num_cores=NC, num_subcores=NS
)


@functools.partial(
    pl.kernel,
    out_type=jax.ShapeDtypeStruct((BATCH, NUM_FIELDS * OUTPUT_DIM), jnp.float32),
    mesh=_mesh,
    scratch_types=[
        pltpu.VMEM((TROWS, 128), jnp.float32),
        pltpu.VMEM((BLK, NUM_FIELDS), jnp.int32),
        pltpu.VMEM((BLK, NUM_FIELDS * OUTPUT_DIM), jnp.float32),
        pltpu.SemaphoreType.DMA,
    ],
    compiler_params=pltpu.CompilerParams(
        use_tc_tiling_on_sc=True, needs_layout_passes=False
    ),
)
def _embed_gather(feat_hbm, table_hbm, out_hbm, tab_v, fv, img, sem):
    wid = lax.axis_index("s") * NC + lax.axis_index("c")
    b0 = wid * B_PER_W
    tcopy = pltpu.async_copy(table_hbm, tab_v, sem)

    lane = lax.iota(jnp.int32, L)

    for blk in range(NBLK):
        pltpu.sync_copy(feat_hbm.at[pl.ds(b0 + blk * BLK, BLK)], fv)
        if blk == 0:
            tcopy.wait()

        def group(g, _):
            rows = lane + g * L
            for f in range(NUM_FIELDS):
                fcol = jnp.full((L,), f, jnp.int32)
                tr = plsc.load_gather(fv, [rows, fcol]) + f * FIELD_SIZE
                prow = lax.shift_right_logical(tr, 2)
                pcol = lax.shift_left(lax.bitwise_and(tr, 3), 5)
                for c in range(OUTPUT_DIM):
                    val = plsc.load_gather(tab_v, [prow, pcol + c])
                    plsc.store_scatter(
                        img,
                        [rows, jnp.full((L,), f * OUTPUT_DIM + c, jnp.int32)],
                        val,
                    )
            return 0

        lax.fori_loop(0, GROUPS, group, 0)
        pltpu.sync_copy(img, out_hbm.at[pl.ds(b0 + blk * BLK, BLK)])


def kernel(input_features, table):
    return _embed_gather(
        input_features.astype(jnp.int32), table.reshape(TROWS, 128)
    )


# speedup vs baseline: 2.1221x; 2.1221x over previous
import functools

import jax
import jax.numpy as jnp
from jax import lax
from jax.experimental import pallas as pl
from jax.experimental.pallas import tpu as pltpu
from jax.experimental.pallas import tpu_sc as plsc

OUTPUT_DIM = 32
NUM_FIELDS = 6
FIELD_SIZE = 100
BATCH = 16384
VOCAB = NUM_FIELDS * FIELD_SIZE

NC, NS, L = 2, 16, 16
NW = NC * NS
B_PER_W = BATCH // NW
BLK = 128
NBLK = B_PER_W // BLK
GROUPS = BLK // L

_mesh = plsc.VectorSubcoreMesh(
    core_axis_name="c", subcore_axis_name="s", num_cores=NC, num_subcores=NS
)


@functools.partial(
    pl.kernel,
    out_type=jax.ShapeDtypeStruct((BATCH, NUM_FIELDS * OUTPUT_DIM), jnp.float32),
    mesh=_mesh,
    scratch_types=[
        pltpu.VMEM_SHARED((VOCAB, 128), jnp.float32),
        pltpu.VMEM((BLK, NUM_FIELDS), jnp.int32),
        pltpu.VMEM((NUM_FIELDS * BLK,), jnp.int32),
        pltpu.VMEM((BLK, 128), jnp.float32),
        pltpu.VMEM((BLK, 128), jnp.float32),
        pltpu.VMEM((BLK, NUM_FIELDS * OUTPUT_DIM), jnp.float32),
        pltpu.SemaphoreType.DMA,
        pltpu.SemaphoreType.DMA,
    ],
    compiler_params=pltpu.CompilerParams(
        use_tc_tiling_on_sc=True, needs_layout_passes=False
    ),
)
def _embed_gather(
    feat_hbm, table_hbm, out_hbm, tab_sh, fv, idx_v, rbuf0, rbuf1, img, sem, semt
):
    wid = lax.axis_index("s") * NC + lax.axis_index("c")
    sid = lax.axis_index("s")
    b0 = wid * B_PER_W

    @pl.when(sid == 0)
    def _():
        pltpu.sync_copy(table_hbm, tab_sh)

    plsc.subcore_barrier()

    lane = lax.iota(jnp.int32, L)

    def assemble(f, buf):
        def row(j, _):
            for h in range(OUTPUT_DIM // L):
                img[j, pl.ds(f * OUTPUT_DIM + h * L, L)] = buf[j, pl.ds(h * L, L)]
            return 0

        lax.fori_loop(0, BLK, row, 0)

    for blk in range(NBLK):
        pltpu.sync_copy(feat_hbm.at[pl.ds(b0 + blk * BLK, BLK)], fv)

        def group(g, _):
            rows = lane + g * L
            for f in range(NUM_FIELDS):
                fcol = jnp.full((L,), f, jnp.int32)
                tr = plsc.load_gather(fv, [rows, fcol]) + f * FIELD_SIZE
                plsc.store_scatter(idx_v, [rows + f * BLK], tr)
            return 0

        lax.fori_loop(0, GROUPS, group, 0)

        bufs = [rbuf0, rbuf1]
        copies = [None, None]
        for f in range(NUM_FIELDS):
            copies[f % 2] = pltpu.async_copy(
                tab_sh.at[idx_v.at[pl.ds(f * BLK, BLK)]],
                bufs[f % 2],
                sem,
            )
            if f > 0:
                copies[(f - 1) % 2].wait()
                assemble(f - 1, bufs[(f - 1) % 2])
        copies[(NUM_FIELDS - 1) % 2].wait()
        assemble(NUM_FIELDS - 1, bufs[(NUM_FIELDS - 1) % 2])

        pltpu.sync_copy(img, out_hbm.at[pl.ds(b0 + blk * BLK, BLK)])


def kernel(input_features, table):
    table_pad = jnp.pad(table, ((0, 0), (0, 128 - OUTPUT_DIM)))
    return _embed_gather(input_features.astype(jnp.int32), table_pad)


# speedup vs baseline: 2.1655x; 1.0205x over previous
import functools

import jax
import jax.numpy as jnp
from jax import lax
from jax.experimental import pallas as pl
from jax.experimental.pallas import tpu as pltpu
from jax.experimental.pallas import tpu_sc as plsc

OUTPUT_DIM = 32
NUM_FIELDS = 6
FIELD_SIZE = 100
BATCH = 16384
VOCAB = NUM_FIELDS * FIELD_SIZE

NC, NS, L = 2, 16, 16
NW = NC * NS
B_PER_W = BATCH // NW
BLK = 128
NBLK = B_PER_W // BLK
GROUPS = BLK // L

_mesh = plsc.VectorSubcoreMesh(
    core_axis_name="c", subcore_axis_name="s", num_cores=NC, num_subcores=NS
)


@functools.partial(
    pl.kernel,
    out_type=jax.ShapeDtypeStruct((BATCH, NUM_FIELDS * OUTPUT_DIM), jnp.float32),
    mesh=_mesh,
    scratch_types=[
        pltpu.VMEM_SHARED((VOCAB, 128), jnp.float32),
        pltpu.VMEM((BLK, NUM_FIELDS), jnp.int32),
        pltpu.VMEM((NUM_FIELDS * BLK,), jnp.int32),
        pltpu.VMEM((BLK, 128), jnp.float32),
        pltpu.VMEM((BLK, 128), jnp.float32),
        pltpu.VMEM((BLK, NUM_FIELDS * OUTPUT_DIM), jnp.float32),
        pltpu.VMEM((BLK, NUM_FIELDS * OUTPUT_DIM), jnp.float32),
        pltpu.SemaphoreType.DMA,
        pltpu.SemaphoreType.DMA,
    ],
    compiler_params=pltpu.CompilerParams(
        use_tc_tiling_on_sc=True, needs_layout_passes=False
    ),
)
def _embed_gather(
    feat_hbm, table_hbm, out_hbm, tab_sh, fv, idx_v, rbuf0, rbuf1, img0, img1,
    sem, semt
):
    wid = lax.axis_index("s") * NC + lax.axis_index("c")
    sid = lax.axis_index("s")
    b0 = wid * B_PER_W

    @pl.when(sid == 0)
    def _():
        pltpu.sync_copy(table_hbm, tab_sh)

    plsc.subcore_barrier()

    lane = lax.iota(jnp.int32, L)

    def assemble(f, buf, img):
        def row(j, _):
            for h in range(OUTPUT_DIM // L):
                img[j, pl.ds(f * OUTPUT_DIM + h * L, L)] = buf[j, pl.ds(h * L, L)]
            return 0

        lax.fori_loop(0, BLK, row, 0)

    imgs = [img0, img1]
    wcopies = [None, None]
    for blk in range(NBLK):
        img = imgs[blk % 2]
        if wcopies[blk % 2] is not None:
            wcopies[blk % 2].wait()
        pltpu.sync_copy(feat_hbm.at[pl.ds(b0 + blk * BLK, BLK)], fv)

        def group(g, _):
            rows = lane + g * L
            for f in range(NUM_FIELDS):
                fcol = jnp.full((L,), f, jnp.int32)
                tr = plsc.load_gather(fv, [rows, fcol]) + f * FIELD_SIZE
                plsc.store_scatter(idx_v, [rows + f * BLK], tr)
            return 0

        lax.fori_loop(0, GROUPS, group, 0)

        bufs = [rbuf0, rbuf1]
        copies = [None, None]
        for f in range(NUM_FIELDS):
            copies[f % 2] = pltpu.async_copy(
                tab_sh.at[idx_v.at[pl.ds(f * BLK, BLK)]],
                bufs[f % 2],
                sem,
            )
            if f > 0:
                copies[(f - 1) % 2].wait()
                assemble(f - 1, bufs[(f - 1) % 2], img)
        copies[(NUM_FIELDS - 1) % 2].wait()
        assemble(NUM_FIELDS - 1, bufs[(NUM_FIELDS - 1) % 2], img)

        wcopies[blk % 2] = pltpu.async_copy(
            img, out_hbm.at[pl.ds(b0 + blk * BLK, BLK)], semt
        )
    for wc in wcopies:
        if wc is not None:
            wc.wait()


def kernel(input_features, table):
    table_pad = jnp.pad(table, ((0, 0), (0, 128 - OUTPUT_DIM)))
    return _embed_gather(input_features.astype(jnp.int32), table_pad)
